# Initial kernel scaffold; baseline (speedup 1.0000x reference)
import jax
import jax.numpy as jnp
from jax.experimental import pallas as pl


def kernel(x, edge_index, edge_weight, eps, W1_0, b1_0, W2_0, b2_0, bnm_g_0, bnm_b_0, bn_g_0, bn_b_0, W1_1, b1_1, W2_1, b2_1, bnm_g_1, bnm_b_1, bn_g_1, bn_b_1):
    raise NotImplementedError("write your pallas kernel here")


# speedup vs baseline: 3.2765x; 3.2765x over previous
import functools

import jax
import jax.numpy as jnp
from jax import lax
from jax.experimental import pallas as pl
from jax.experimental.pallas import tpu as pltpu
from jax.experimental.pallas import tpu_sc as plsc

N = 10000
D = 128
E = 320000

NC = 2
NS = 16
NW = NC * NS
CH = 128
NCHUNK = 79
EPT = NCHUNK * CH
E2 = NW * EPT
NPAD = 10112
ZR = NPAD // NS
ZH = 8


def _agg_body(h_hbm, src_hbm, dst_hbm, w_hbm, out_hbm,
              src_c, dst_c, w_c, rows_v, zbuf_v, agg_sh, sem):
    cid = lax.axis_index("c")
    sid = lax.axis_index("s")
    wid = cid * NS + sid

    for r in range(ZH):
        for c in range(D // 16):
            zbuf_v[r, pl.ds(c * 16, 16)] = jnp.zeros((16,), jnp.float32)

    def _zcopy(k, _):
        pltpu.sync_copy(zbuf_v, agg_sh.at[pl.ds(sid * ZR + k * ZH, ZH)])
        return 0
    lax.fori_loop(0, ZR // ZH, _zcopy, 0)

    plsc.subcore_barrier()

    def _chunk(j, _):
        base = (wid * NCHUNK + j) * CH
        pltpu.sync_copy(src_hbm.at[pl.ds(base, CH)], src_c)
        pltpu.sync_copy(dst_hbm.at[pl.ds(base, CH)], dst_c)
        pltpu.sync_copy(w_hbm.at[pl.ds(base, CH)], w_c)

        pltpu.async_copy(h_hbm.at[src_c], rows_v, sem).wait()

        def _grp(g, _):
            wvec = w_c[pl.ds(g * 16, 16)]
            for e16 in range(16):
                w = wvec[e16]
                e = g * 16 + e16
                for r in range(D // 16):
                    sl = pl.ds(r * 16, 16)
                    rows_v[e, sl] = rows_v[e, sl] * w
            return 0
        lax.fori_loop(0, CH // 16, _grp, 0)

        pltpu.sync_copy(rows_v, agg_sh.at[dst_c], add=True)
        return 0
    lax.fori_loop(0, NCHUNK, _chunk, 0)

    plsc.subcore_barrier()

    pltpu.sync_copy(agg_sh.at[pl.ds(sid * ZR, ZR)],
                    out_hbm.at[cid, pl.ds(sid * ZR, ZR)])


@jax.jit
def _agg(h, src, dst, w):
    mesh = plsc.VectorSubcoreMesh(core_axis_name="c", subcore_axis_name="s")
    return pl.kernel(
        _agg_body,
        out_type=jax.ShapeDtypeStruct((NC, NPAD, D), jnp.float32),
        mesh=mesh,
        scratch_types=[
            pltpu.VMEM((CH,), jnp.int32),
            pltpu.VMEM((CH,), jnp.int32),
            pltpu.VMEM((CH,), jnp.float32),
            pltpu.VMEM((CH, D), jnp.float32),
            pltpu.VMEM((ZH, D), jnp.float32),
            pltpu.VMEM_SHARED((NPAD, D), jnp.float32),
            pltpu.SemaphoreType.DMA,
        ],
    )(h, src, dst, w)


def _mlp_body(h_ref, agg_ref, eps_ref, W1_ref, b1_ref, W2_ref, b2_ref,
              g1_ref, B1_ref, g2_ref, B2_ref, out_ref):
    h = h_ref[...]
    agg = agg_ref[0, :N, :] + agg_ref[1, :N, :]
    hp = (1.0 + eps_ref[0, 0]) * h + agg
    y = jnp.dot(hp, W1_ref[...], preferred_element_type=jnp.float32) + b1_ref[...]
    m = jnp.mean(y, axis=0, keepdims=True)
    v = jnp.mean((y - m) ** 2, axis=0, keepdims=True)
    y = g1_ref[...] * (y - m) * lax.rsqrt(v + 1e-5) + B1_ref[...]
    y = jnp.maximum(y, 0.0)
    y = jnp.dot(y, W2_ref[...], preferred_element_type=jnp.float32) + b2_ref[...]
    m = jnp.mean(y, axis=0, keepdims=True)
    v = jnp.mean((y - m) ** 2, axis=0, keepdims=True)
    y = g2_ref[...] * (y - m) * lax.rsqrt(v + 1e-5) + B2_ref[...]
    out_ref[...] = jnp.maximum(y, 0.0)


@jax.jit
def _mlp(h, agg, eps_l, W1, b1, W2, b2, g1, B1, g2, B2):
    vmem = pl.BlockSpec(memory_space=pltpu.VMEM)
    return pl.pallas_call(
        _mlp_body,
        out_shape=jax.ShapeDtypeStruct((N, D), jnp.float32),
        in_specs=[vmem, vmem, pl.BlockSpec(memory_space=pltpu.SMEM)] + [vmem] * 8,
        out_specs=vmem,
    )(h, agg, eps_l, W1, b1, W2, b2, g1, B1, g2, B2)


def kernel(x, edge_index, edge_weight, eps,
           W1_0, b1_0, W2_0, b2_0, bnm_g_0, bnm_b_0, bn_g_0, bn_b_0,
           W1_1, b1_1, W2_1, b2_1, bnm_g_1, bnm_b_1, bn_g_1, bn_b_1):
    pad = E2 - E
    src = jnp.concatenate([edge_index[0], jnp.zeros((pad,), jnp.int32)])
    dst = jnp.concatenate([edge_index[1], jnp.zeros((pad,), jnp.int32)])
    w = jnp.concatenate([edge_weight, jnp.zeros((pad,), jnp.float32)])
    layers = [
        (W1_0, b1_0, W2_0, b2_0, bnm_g_0, bnm_b_0, bn_g_0, bn_b_0),
        (W1_1, b1_1, W2_1, b2_1, bnm_g_1, bnm_b_1, bn_g_1, bn_b_1),
    ]
    h = x
    for l in range(2):
        W1, b1, W2, b2, g1, B1, g2, B2 = layers[l]
        agg = _agg(h, src, dst, w)
        h = _mlp(h, agg, eps[l].reshape(1, 1),
                 W1, b1.reshape(1, D), W2, b2.reshape(1, D),
                 g1.reshape(1, D), B1.reshape(1, D),
                 g2.reshape(1, D), B2.reshape(1, D))
    return h
